# Initial kernel scaffold; baseline (speedup 1.0000x reference)
import jax
import jax.numpy as jnp
from jax.experimental import pallas as pl


def kernel(query, reference_points, input_flatten, input_spatial_shapes, input_level_start_index, W_off, b_off, W_attn, b_attn, W_val, b_val, W_out, b_out):
    raise NotImplementedError("write your pallas kernel here")


# speedup vs baseline: 1023.6170x; 1023.6170x over previous
import functools

import numpy as np
import jax
import jax.numpy as jnp
from jax import lax
from jax.experimental import pallas as pl
from jax.experimental.pallas import tpu as pltpu
from jax.experimental.pallas import tpu_sc as plsc

NH = 8
NL = 4
NP = 4
DM = 256
HD = 32
LP = NL * NP
LVL_SHAPES = [(48, 48), (24, 24), (12, 12), (6, 6)]
LEN = sum(h * w for h, w in LVL_SHAPES)
LEN_P = 3072
BS = 4
QB = 768
NBLK = LEN_P // QB
QC = 48
NCH = LEN_P // QC

_l_of_col = (np.arange(NH * LP) // NP) % NL
_W_f = np.array([w for h, w in LVL_SHAPES], np.float32)
_H_f = np.array([h for h, w in LVL_SHAPES], np.float32)
_hw = np.array([h * w for h, w in LVL_SHAPES], np.int64)
_start = np.concatenate([[0], np.cumsum(_hw)[:-1]]).astype(np.int32)
_cf = np.zeros((8, NH * LP), np.float32)
_cf[_l_of_col, np.arange(NH * LP)] = 1.0
_cf[4] = _W_f[_l_of_col]
_cf[5] = _H_f[_l_of_col]
CONST_F = _cf
_ci = np.zeros((8, NH * LP), np.int32)
_ci[0] = _W_f[_l_of_col].astype(np.int32)
_ci[1] = _H_f[_l_of_col].astype(np.int32)
_ci[2] = _start[_l_of_col]
CONST_I = _ci


def _prep_body(q_ref, x_ref, rpx_ref, rpy_ref, wval_ref, bval_ref,
               wx_ref, bx_ref, wy_ref, by_ref, wa_ref, ba_ref,
               cf_ref, ci_ref, val_out, idx_out, w_out):
    f32 = jnp.float32
    cd = (((1,), (1,)), ((), ()))
    q = q_ref[0]
    x_in = x_ref[0]
    val = (lax.dot_general(x_in, wval_ref[...], cd,
                           preferred_element_type=f32, precision=lax.Precision.HIGHEST)
           + bval_ref[...])
    val_out[0] = val.reshape(QB, NH, HD).transpose(1, 0, 2)
    offx = lax.dot_general(q, wx_ref[...], cd, preferred_element_type=f32, precision=lax.Precision.HIGHEST) + bx_ref[...]
    offy = lax.dot_general(q, wy_ref[...], cd, preferred_element_type=f32, precision=lax.Precision.HIGHEST) + by_ref[...]
    logits = lax.dot_general(q, wa_ref[...], cd, preferred_element_type=f32, precision=lax.Precision.HIGHEST) + ba_ref[...]
    l3 = logits.reshape(QB, NH, LP)
    l3 = l3 - jnp.max(l3, axis=2, keepdims=True)
    e = jnp.exp(l3)
    aw = (e / jnp.sum(e, axis=2, keepdims=True)).reshape(QB, NH * LP)

    sel = cf_ref[0:NL]
    col_w = cf_ref[4:5]
    col_h = cf_ref[5:6]
    col_wi = ci_ref[0:1]
    col_hi = ci_ref[1:2]
    col_start = ci_ref[2:3]
    cd0 = (((1,), (0,)), ((), ()))
    rx = lax.dot_general(rpx_ref[0], sel, cd0, preferred_element_type=f32, precision=lax.Precision.HIGHEST)
    ry = lax.dot_general(rpy_ref[0], sel, cd0, preferred_element_type=f32, precision=lax.Precision.HIGHEST)
    x = rx * col_w + offx - 0.5
    y = ry * col_h + offy - 0.5
    x0f = jnp.floor(x)
    y0f = jnp.floor(y)
    fx = x - x0f
    fy = y - y0f
    ix0 = x0f.astype(jnp.int32)
    iy0 = y0f.astype(jnp.int32)
    one = jnp.float32(1.0)
    vx0 = ((x0f >= 0) & (x0f <= col_w - 1)).astype(f32)
    vx1 = ((x0f + 1 >= 0) & (x0f + 1 <= col_w - 1)).astype(f32)
    vy0 = ((y0f >= 0) & (y0f <= col_h - 1)).astype(f32)
    vy1 = ((y0f + 1 >= 0) & (y0f + 1 <= col_h - 1)).astype(f32)
    cx0 = jnp.clip(ix0, 0, col_wi - 1)
    cx1 = jnp.clip(ix0 + 1, 0, col_wi - 1)
    cy0 = jnp.clip(iy0, 0, col_hi - 1)
    cy1 = jnp.clip(iy0 + 1, 0, col_hi - 1)
    r00 = (col_start + cy0 * col_wi + cx0) * HD
    r01 = (col_start + cy0 * col_wi + cx1) * HD
    r10 = (col_start + cy1 * col_wi + cx0) * HD
    r11 = (col_start + cy1 * col_wi + cx1) * HD
    w00 = (one - fx) * (one - fy) * vx0 * vy0 * aw
    w01 = fx * (one - fy) * vx1 * vy0 * aw
    w10 = (one - fx) * fy * vx0 * vy1 * aw
    w11 = fx * fy * vx1 * vy1 * aw
    idx4 = jnp.stack([r00, r01, r10, r11], axis=0).reshape(4, QB, NH, LP)
    w4 = jnp.stack([w00, w01, w10, w11], axis=0).reshape(4, QB, NH, LP)
    idx_out[0] = idx4.transpose(2, 1, 0, 3).reshape(NH, QB, 4 * LP)
    w_out[0] = w4.transpose(2, 1, 0, 3).reshape(NH, QB, 4 * LP)


_prep_call = pl.pallas_call(
    _prep_body,
    grid=(BS, NBLK),
    in_specs=[
        pl.BlockSpec((1, QB, DM), lambda b, i: (b, i, 0)),
        pl.BlockSpec((1, QB, DM), lambda b, i: (b, i, 0)),
        pl.BlockSpec((1, QB, NL), lambda b, i: (b, i, 0)),
        pl.BlockSpec((1, QB, NL), lambda b, i: (b, i, 0)),
        pl.BlockSpec((DM, DM), lambda b, i: (0, 0)),
        pl.BlockSpec((1, DM), lambda b, i: (0, 0)),
        pl.BlockSpec((NH * LP, DM), lambda b, i: (0, 0)),
        pl.BlockSpec((1, NH * LP), lambda b, i: (0, 0)),
        pl.BlockSpec((NH * LP, DM), lambda b, i: (0, 0)),
        pl.BlockSpec((1, NH * LP), lambda b, i: (0, 0)),
        pl.BlockSpec((NH * LP, DM), lambda b, i: (0, 0)),
        pl.BlockSpec((1, NH * LP), lambda b, i: (0, 0)),
        pl.BlockSpec((8, NH * LP), lambda b, i: (0, 0)),
        pl.BlockSpec((8, NH * LP), lambda b, i: (0, 0)),
    ],
    out_specs=[
        pl.BlockSpec((1, NH, QB, HD), lambda b, i: (b, 0, i, 0)),
        pl.BlockSpec((1, NH, QB, 4 * LP), lambda b, i: (b, 0, i, 0)),
        pl.BlockSpec((1, NH, QB, 4 * LP), lambda b, i: (b, 0, i, 0)),
    ],
    out_shape=[
        jax.ShapeDtypeStruct((BS, NH, LEN_P, HD), jnp.float32),
        jax.ShapeDtypeStruct((BS, NH, LEN_P, 4 * LP), jnp.int32),
        jax.ShapeDtypeStruct((BS, NH, LEN_P, 4 * LP), jnp.float32),
    ],
)


def _sc_body(value_hbm, idx_hbm, w_hbm, out_hbm,
             plane_v, idx_v, w_v, out_v, sem_pl, sem_in, sem_out):
    cid = lax.axis_index("c")
    sid = lax.axis_index("s")
    wid = sid * 2 + cid
    b = wid // NH
    h = wid % NH
    plane_words = LEN_P * HD
    pltpu.async_copy(
        value_hbm.at[pl.ds(wid * plane_words, plane_words)], plane_v, sem_pl
    ).wait()
    iota = lax.iota(jnp.int32, 16)
    iota_hi = iota + 16

    def chunk_body(ci, carry):
        qoff = ci * QC
        c1 = pltpu.async_copy(idx_hbm.at[b, h, pl.ds(qoff, QC)], idx_v, sem_in)
        c2 = pltpu.async_copy(w_hbm.at[b, h, pl.ds(qoff, QC)], w_v, sem_in)
        c1.wait()
        c2.wait()

        def q_body(qi, c):
            accs = [jnp.zeros((16,), jnp.float32) for _ in range(8)]
            for k in range(4):
                ivec = idx_v[qi, pl.ds(k * LP, LP)]
                wvec = w_v[qi, pl.ds(k * LP, LP)]
                for j in range(LP):
                    wv = jnp.full((16,), wvec[j], jnp.float32)
                    rv = jnp.full((16,), ivec[j], jnp.int32)
                    g0 = plsc.load_gather(plane_v, [rv + iota])
                    g1 = plsc.load_gather(plane_v, [rv + iota_hi])
                    accs[2 * k] = accs[2 * k] + wv * g0
                    accs[2 * k + 1] = accs[2 * k + 1] + wv * g1
            out_v[qi, pl.ds(0, 16)] = (accs[0] + accs[2]) + (accs[4] + accs[6])
            out_v[qi, pl.ds(16, 16)] = (accs[1] + accs[3]) + (accs[5] + accs[7])
            return c

        lax.fori_loop(0, QC, q_body, 0)
        pltpu.async_copy(out_v, out_hbm.at[b, h, pl.ds(qoff, QC)], sem_out).wait()
        return carry

    lax.fori_loop(0, NCH, chunk_body, 0)


@functools.cache
def _get_sc_call():
    return pl.kernel(
        _sc_body,
        out_type=jax.ShapeDtypeStruct((BS, NH, LEN_P, HD), jnp.float32),
        mesh=plsc.VectorSubcoreMesh(core_axis_name="c", subcore_axis_name="s"),
        compiler_params=pltpu.CompilerParams(needs_layout_passes=False),
        scratch_types=[
            pltpu.VMEM((LEN_P * HD,), jnp.float32),
            pltpu.VMEM((QC, 4 * LP), jnp.int32),
            pltpu.VMEM((QC, 4 * LP), jnp.float32),
            pltpu.VMEM((QC, HD), jnp.float32),
            pltpu.SemaphoreType.DMA,
            pltpu.SemaphoreType.DMA,
            pltpu.SemaphoreType.DMA,
        ],
    )


def _out_body(a_ref, w_ref, b_ref, o_ref):
    cd = (((1,), (1,)), ((), ()))
    a = a_ref[0]
    acc = b_ref[...]
    w = w_ref[...]
    outs = []
    for h in range(NH):
        wh = w[:, h * HD:(h + 1) * HD]
        outs.append(lax.dot_general(a[h], wh, cd,
                                    preferred_element_type=jnp.float32, precision=lax.Precision.HIGHEST))
    o_ref[0] = ((outs[0] + outs[1]) + (outs[2] + outs[3])
                + (outs[4] + outs[5]) + (outs[6] + outs[7]) + acc)


_out_call = pl.pallas_call(
    _out_body,
    grid=(BS, NBLK),
    in_specs=[
        pl.BlockSpec((1, NH, QB, HD), lambda b, i: (b, 0, i, 0)),
        pl.BlockSpec((DM, DM), lambda b, i: (0, 0)),
        pl.BlockSpec((1, DM), lambda b, i: (0, 0)),
    ],
    out_specs=pl.BlockSpec((1, QB, DM), lambda b, i: (b, i, 0)),
    out_shape=jax.ShapeDtypeStruct((BS, LEN_P, DM), jnp.float32),
)


def kernel(query, reference_points, input_flatten, input_spatial_shapes,
           input_level_start_index, W_off, b_off, W_attn, b_attn,
           W_val, b_val, W_out, b_out):
    del input_spatial_shapes, input_level_start_index
    Wx = W_off[0::2]
    Wy = W_off[1::2]
    bx = b_off[0::2].reshape(1, NH * LP)
    by = b_off[1::2].reshape(1, NH * LP)
    pad = ((0, 0), (0, LEN_P - LEN), (0, 0))
    query_p = jnp.pad(query, pad)
    xf_p = jnp.pad(input_flatten, pad)
    rpx = jnp.pad(reference_points[..., 0], pad)
    rpy = jnp.pad(reference_points[..., 1], pad)
    value, idxs, ws = _prep_call(
        query_p, xf_p, rpx, rpy, W_val, b_val.reshape(1, DM),
        Wx, bx, Wy, by, W_attn, b_attn.reshape(1, NH * LP),
        jnp.asarray(CONST_F), jnp.asarray(CONST_I))
    attn = _get_sc_call()(value.reshape(-1), idxs, ws)
    return _out_call(attn, W_out, b_out.reshape(1, DM))[:, :LEN]


# speedup vs baseline: 1161.4610x; 1.1347x over previous
import functools

import numpy as np
import jax
import jax.numpy as jnp
from jax import lax
from jax.experimental import pallas as pl
from jax.experimental.pallas import tpu as pltpu
from jax.experimental.pallas import tpu_sc as plsc

NH = 8
NL = 4
NP = 4
DM = 256
HD = 32
LP = NL * NP
LVL_SHAPES = [(48, 48), (24, 24), (12, 12), (6, 6)]
LEN = sum(h * w for h, w in LVL_SHAPES)
LEN_P = 3072
BS = 4
QB = 768
NBLK = LEN_P // QB
QC = 48
NCH = LEN_P // QC

_l_of_col = (np.arange(NH * LP) // NP) % NL
_W_f = np.array([w for h, w in LVL_SHAPES], np.float32)
_H_f = np.array([h for h, w in LVL_SHAPES], np.float32)
_hw = np.array([h * w for h, w in LVL_SHAPES], np.int64)
_start = np.concatenate([[0], np.cumsum(_hw)[:-1]]).astype(np.int32)
_cf = np.zeros((8, NH * LP), np.float32)
_cf[_l_of_col, np.arange(NH * LP)] = 1.0
_cf[4] = _W_f[_l_of_col]
_cf[5] = _H_f[_l_of_col]
CONST_F = _cf
_ci = np.zeros((8, NH * LP), np.int32)
_ci[0] = _W_f[_l_of_col].astype(np.int32)
_ci[1] = _H_f[_l_of_col].astype(np.int32)
_ci[2] = _start[_l_of_col]
CONST_I = _ci


def _prep_body(q_ref, x_ref, rpx_ref, rpy_ref, wval_ref, bval_ref,
               wx_ref, bx_ref, wy_ref, by_ref, wa_ref, ba_ref,
               cf_ref, ci_ref, val_out, idx_out, w_out):
    f32 = jnp.float32
    cd = (((1,), (1,)), ((), ()))
    q = q_ref[0]
    x_in = x_ref[0]
    val = (lax.dot_general(x_in, wval_ref[...], cd,
                           preferred_element_type=f32, precision=lax.Precision.HIGHEST)
           + bval_ref[...])
    val_out[0] = val.reshape(QB, NH, HD).transpose(1, 0, 2)
    offx = lax.dot_general(q, wx_ref[...], cd, preferred_element_type=f32, precision=lax.Precision.HIGHEST) + bx_ref[...]
    offy = lax.dot_general(q, wy_ref[...], cd, preferred_element_type=f32, precision=lax.Precision.HIGHEST) + by_ref[...]
    logits = lax.dot_general(q, wa_ref[...], cd, preferred_element_type=f32, precision=lax.Precision.HIGHEST) + ba_ref[...]
    l3 = logits.reshape(QB, NH, LP)
    l3 = l3 - jnp.max(l3, axis=2, keepdims=True)
    e = jnp.exp(l3)
    aw = (e / jnp.sum(e, axis=2, keepdims=True)).reshape(QB, NH * LP)

    sel = cf_ref[0:NL]
    col_w = cf_ref[4:5]
    col_h = cf_ref[5:6]
    col_wi = ci_ref[0:1]
    col_hi = ci_ref[1:2]
    col_start = ci_ref[2:3]
    cd0 = (((1,), (0,)), ((), ()))
    rx = lax.dot_general(rpx_ref[0], sel, cd0, preferred_element_type=f32, precision=lax.Precision.HIGHEST)
    ry = lax.dot_general(rpy_ref[0], sel, cd0, preferred_element_type=f32, precision=lax.Precision.HIGHEST)
    x = rx * col_w + offx - 0.5
    y = ry * col_h + offy - 0.5
    x0f = jnp.floor(x)
    y0f = jnp.floor(y)
    fx = x - x0f
    fy = y - y0f
    ix0 = x0f.astype(jnp.int32)
    iy0 = y0f.astype(jnp.int32)
    one = jnp.float32(1.0)
    vx0 = ((x0f >= 0) & (x0f <= col_w - 1)).astype(f32)
    vx1 = ((x0f + 1 >= 0) & (x0f + 1 <= col_w - 1)).astype(f32)
    vy0 = ((y0f >= 0) & (y0f <= col_h - 1)).astype(f32)
    vy1 = ((y0f + 1 >= 0) & (y0f + 1 <= col_h - 1)).astype(f32)
    cx0 = jnp.clip(ix0, 0, col_wi - 1)
    cx1 = jnp.clip(ix0 + 1, 0, col_wi - 1)
    cy0 = jnp.clip(iy0, 0, col_hi - 1)
    cy1 = jnp.clip(iy0 + 1, 0, col_hi - 1)
    r00 = (col_start + cy0 * col_wi + cx0) * HD
    r01 = (col_start + cy0 * col_wi + cx1) * HD
    r10 = (col_start + cy1 * col_wi + cx0) * HD
    r11 = (col_start + cy1 * col_wi + cx1) * HD
    w00 = (one - fx) * (one - fy) * vx0 * vy0 * aw
    w01 = fx * (one - fy) * vx1 * vy0 * aw
    w10 = (one - fx) * fy * vx0 * vy1 * aw
    w11 = fx * fy * vx1 * vy1 * aw
    idx4 = jnp.stack([r00, r01, r10, r11], axis=0).reshape(4, QB, NH, LP)
    w4 = jnp.stack([w00, w01, w10, w11], axis=0).reshape(4, QB, NH, LP)
    idx_out[0] = idx4.transpose(2, 1, 0, 3).reshape(NH, QB, 4 * LP)
    w_out[0] = w4.transpose(2, 1, 0, 3).reshape(NH, QB, 4 * LP)


_prep_call = pl.pallas_call(
    _prep_body,
    grid=(BS, NBLK),
    in_specs=[
        pl.BlockSpec((1, QB, DM), lambda b, i: (b, i, 0)),
        pl.BlockSpec((1, QB, DM), lambda b, i: (b, i, 0)),
        pl.BlockSpec((1, QB, NL), lambda b, i: (b, i, 0)),
        pl.BlockSpec((1, QB, NL), lambda b, i: (b, i, 0)),
        pl.BlockSpec((DM, DM), lambda b, i: (0, 0)),
        pl.BlockSpec((1, DM), lambda b, i: (0, 0)),
        pl.BlockSpec((NH * LP, DM), lambda b, i: (0, 0)),
        pl.BlockSpec((1, NH * LP), lambda b, i: (0, 0)),
        pl.BlockSpec((NH * LP, DM), lambda b, i: (0, 0)),
        pl.BlockSpec((1, NH * LP), lambda b, i: (0, 0)),
        pl.BlockSpec((NH * LP, DM), lambda b, i: (0, 0)),
        pl.BlockSpec((1, NH * LP), lambda b, i: (0, 0)),
        pl.BlockSpec((8, NH * LP), lambda b, i: (0, 0)),
        pl.BlockSpec((8, NH * LP), lambda b, i: (0, 0)),
    ],
    out_specs=[
        pl.BlockSpec((1, NH, QB, HD), lambda b, i: (b, 0, i, 0)),
        pl.BlockSpec((1, NH, QB, 4 * LP), lambda b, i: (b, 0, i, 0)),
        pl.BlockSpec((1, NH, QB, 4 * LP), lambda b, i: (b, 0, i, 0)),
    ],
    out_shape=[
        jax.ShapeDtypeStruct((BS, NH, LEN_P, HD), jnp.float32),
        jax.ShapeDtypeStruct((BS, NH, LEN_P, 4 * LP), jnp.int32),
        jax.ShapeDtypeStruct((BS, NH, LEN_P, 4 * LP), jnp.float32),
    ],
)


def _sc_body(value_hbm, idx_hbm, w_hbm, out_hbm,
             plane_v, idx_v, w_v, out_v, sem_pl, sem_in, sem_out):
    cid = lax.axis_index("c")
    sid = lax.axis_index("s")
    wid = sid * 2 + cid
    b = wid // NH
    h = wid % NH
    plane_words = LEN_P * HD
    pltpu.async_copy(
        value_hbm.at[pl.ds(wid * plane_words, plane_words)], plane_v, sem_pl
    ).wait()
    iota = lax.iota(jnp.int32, 16)
    iota_hi = iota + 16

    def chunk_body(ci, carry):
        qoff = ci * QC
        c1 = pltpu.async_copy(idx_hbm.at[b, h, pl.ds(qoff, QC)], idx_v, sem_in)
        c2 = pltpu.async_copy(w_hbm.at[b, h, pl.ds(qoff, QC)], w_v, sem_in)
        c1.wait()
        c2.wait()

        lane_consts = [jnp.full((16,), j, jnp.int32) for j in range(LP)]

        def q_body(qi, c):
            accs = [jnp.zeros((16,), jnp.float32) for _ in range(8)]
            for k in range(4):
                ivec = idx_v[qi, pl.ds(k * LP, LP)]
                wvec = w_v[qi, pl.ds(k * LP, LP)]
                for j in range(LP):
                    rv = jnp.take_along_axis(
                        ivec, lane_consts[j], axis=0, mode="promise_in_bounds")
                    wv = jnp.take_along_axis(
                        wvec, lane_consts[j], axis=0, mode="promise_in_bounds")
                    g0 = plsc.load_gather(plane_v, [rv + iota])
                    g1 = plsc.load_gather(plane_v, [rv + iota_hi])
                    accs[2 * k] = accs[2 * k] + wv * g0
                    accs[2 * k + 1] = accs[2 * k + 1] + wv * g1
            out_v[qi, pl.ds(0, 16)] = (accs[0] + accs[2]) + (accs[4] + accs[6])
            out_v[qi, pl.ds(16, 16)] = (accs[1] + accs[3]) + (accs[5] + accs[7])
            return c

        lax.fori_loop(0, QC, q_body, 0)
        pltpu.async_copy(out_v, out_hbm.at[b, h, pl.ds(qoff, QC)], sem_out).wait()
        return carry

    lax.fori_loop(0, NCH, chunk_body, 0)


@functools.cache
def _get_sc_call():
    return pl.kernel(
        _sc_body,
        out_type=jax.ShapeDtypeStruct((BS, NH, LEN_P, HD), jnp.float32),
        mesh=plsc.VectorSubcoreMesh(core_axis_name="c", subcore_axis_name="s"),
        compiler_params=pltpu.CompilerParams(needs_layout_passes=False),
        scratch_types=[
            pltpu.VMEM((LEN_P * HD,), jnp.float32),
            pltpu.VMEM((QC, 4 * LP), jnp.int32),
            pltpu.VMEM((QC, 4 * LP), jnp.float32),
            pltpu.VMEM((QC, HD), jnp.float32),
            pltpu.SemaphoreType.DMA,
            pltpu.SemaphoreType.DMA,
            pltpu.SemaphoreType.DMA,
        ],
    )


def _out_body(a_ref, w_ref, b_ref, o_ref):
    cd = (((1,), (1,)), ((), ()))
    a = a_ref[0]
    acc = b_ref[...]
    w = w_ref[...]
    outs = []
    for h in range(NH):
        wh = w[:, h * HD:(h + 1) * HD]
        outs.append(lax.dot_general(a[h], wh, cd,
                                    preferred_element_type=jnp.float32, precision=lax.Precision.HIGHEST))
    o_ref[0] = ((outs[0] + outs[1]) + (outs[2] + outs[3])
                + (outs[4] + outs[5]) + (outs[6] + outs[7]) + acc)


_out_call = pl.pallas_call(
    _out_body,
    grid=(BS, NBLK),
    in_specs=[
        pl.BlockSpec((1, NH, QB, HD), lambda b, i: (b, 0, i, 0)),
        pl.BlockSpec((DM, DM), lambda b, i: (0, 0)),
        pl.BlockSpec((1, DM), lambda b, i: (0, 0)),
    ],
    out_specs=pl.BlockSpec((1, QB, DM), lambda b, i: (b, i, 0)),
    out_shape=jax.ShapeDtypeStruct((BS, LEN_P, DM), jnp.float32),
)


def kernel(query, reference_points, input_flatten, input_spatial_shapes,
           input_level_start_index, W_off, b_off, W_attn, b_attn,
           W_val, b_val, W_out, b_out):
    del input_spatial_shapes, input_level_start_index
    Wx = W_off[0::2]
    Wy = W_off[1::2]
    bx = b_off[0::2].reshape(1, NH * LP)
    by = b_off[1::2].reshape(1, NH * LP)
    pad = ((0, 0), (0, LEN_P - LEN), (0, 0))
    query_p = jnp.pad(query, pad)
    xf_p = jnp.pad(input_flatten, pad)
    rpx = jnp.pad(reference_points[..., 0], pad)
    rpy = jnp.pad(reference_points[..., 1], pad)
    value, idxs, ws = _prep_call(
        query_p, xf_p, rpx, rpy, W_val, b_val.reshape(1, DM),
        Wx, bx, Wy, by, W_attn, b_attn.reshape(1, NH * LP),
        jnp.asarray(CONST_F), jnp.asarray(CONST_I))
    attn = _get_sc_call()(value.reshape(-1), idxs, ws)
    return _out_call(attn, W_out, b_out.reshape(1, DM))[:, :LEN]


# speedup vs baseline: 2715.1774x; 2.3377x over previous
import functools

import numpy as np
import jax
import jax.numpy as jnp
from jax import lax
from jax.experimental import pallas as pl
from jax.experimental.pallas import tpu as pltpu
from jax.experimental.pallas import tpu_sc as plsc

NH = 8
NL = 4
NP = 4
DM = 256
HD = 32
LP = NL * NP
LVL_SHAPES = [(48, 48), (24, 24), (12, 12), (6, 6)]
LEN = sum(h * w for h, w in LVL_SHAPES)
LEN_P = 3072
BS = 4
QB = 768
NBLK = LEN_P // QB
QC = 48
NCH = LEN_P // QC

_l_of_col = (np.arange(NH * LP) // NP) % NL
_W_f = np.array([w for h, w in LVL_SHAPES], np.float32)
_H_f = np.array([h for h, w in LVL_SHAPES], np.float32)
_hw = np.array([h * w for h, w in LVL_SHAPES], np.int64)
_start = np.concatenate([[0], np.cumsum(_hw)[:-1]]).astype(np.int32)
_cf = np.zeros((8, NH * LP), np.float32)
_cf[_l_of_col, np.arange(NH * LP)] = 1.0
_cf[4] = _W_f[_l_of_col]
_cf[5] = _H_f[_l_of_col]
CONST_F = _cf
_ci = np.zeros((8, NH * LP), np.int32)
_ci[0] = _W_f[_l_of_col].astype(np.int32)
_ci[1] = _H_f[_l_of_col].astype(np.int32)
_ci[2] = _start[_l_of_col]
CONST_I = _ci


def _prep_body(q_ref, x_ref, rpx_ref, rpy_ref, wval_ref, bval_ref,
               wx_ref, bx_ref, wy_ref, by_ref, wa_ref, ba_ref,
               cf_ref, ci_ref, val_out, idx_out, w_out):
    f32 = jnp.float32
    cd = (((1,), (1,)), ((), ()))
    q = q_ref[0]
    x_in = x_ref[0]
    val = (lax.dot_general(x_in, wval_ref[...], cd,
                           preferred_element_type=f32, precision=lax.Precision.HIGHEST)
           + bval_ref[...])
    val_out[0] = val.reshape(QB, NH, HD).transpose(1, 0, 2)
    offx = lax.dot_general(q, wx_ref[...], cd, preferred_element_type=f32, precision=lax.Precision.HIGHEST) + bx_ref[...]
    offy = lax.dot_general(q, wy_ref[...], cd, preferred_element_type=f32, precision=lax.Precision.HIGHEST) + by_ref[...]
    logits = lax.dot_general(q, wa_ref[...], cd, preferred_element_type=f32, precision=lax.Precision.HIGHEST) + ba_ref[...]
    l3 = logits.reshape(QB, NH, LP)
    l3 = l3 - jnp.max(l3, axis=2, keepdims=True)
    e = jnp.exp(l3)
    aw = (e / jnp.sum(e, axis=2, keepdims=True)).reshape(QB, NH * LP)

    sel = cf_ref[0:NL]
    col_w = cf_ref[4:5]
    col_h = cf_ref[5:6]
    col_wi = ci_ref[0:1]
    col_hi = ci_ref[1:2]
    col_start = ci_ref[2:3]
    cd0 = (((1,), (0,)), ((), ()))
    rx = lax.dot_general(rpx_ref[0], sel, cd0, preferred_element_type=f32, precision=lax.Precision.HIGHEST)
    ry = lax.dot_general(rpy_ref[0], sel, cd0, preferred_element_type=f32, precision=lax.Precision.HIGHEST)
    x = rx * col_w + offx - 0.5
    y = ry * col_h + offy - 0.5
    x0f = jnp.floor(x)
    y0f = jnp.floor(y)
    fx = x - x0f
    fy = y - y0f
    ix0 = x0f.astype(jnp.int32)
    iy0 = y0f.astype(jnp.int32)
    one = jnp.float32(1.0)
    vx0 = ((x0f >= 0) & (x0f <= col_w - 1)).astype(f32)
    vx1 = ((x0f + 1 >= 0) & (x0f + 1 <= col_w - 1)).astype(f32)
    vy0 = ((y0f >= 0) & (y0f <= col_h - 1)).astype(f32)
    vy1 = ((y0f + 1 >= 0) & (y0f + 1 <= col_h - 1)).astype(f32)
    cx0 = jnp.clip(ix0, 0, col_wi - 1)
    cx1 = jnp.clip(ix0 + 1, 0, col_wi - 1)
    cy0 = jnp.clip(iy0, 0, col_hi - 1)
    cy1 = jnp.clip(iy0 + 1, 0, col_hi - 1)
    r00 = (col_start + cy0 * col_wi + cx0) * HD
    r01 = (col_start + cy0 * col_wi + cx1) * HD
    r10 = (col_start + cy1 * col_wi + cx0) * HD
    r11 = (col_start + cy1 * col_wi + cx1) * HD
    w00 = (one - fx) * (one - fy) * vx0 * vy0 * aw
    w01 = fx * (one - fy) * vx1 * vy0 * aw
    w10 = (one - fx) * fy * vx0 * vy1 * aw
    w11 = fx * fy * vx1 * vy1 * aw
    idx4 = jnp.stack([r00, r01, r10, r11], axis=0).reshape(4, QB, NH, LP)
    w4 = jnp.stack([w00, w01, w10, w11], axis=0).reshape(4, QB, NH, LP)
    idx_out[0] = idx4.transpose(2, 1, 0, 3).reshape(NH, QB, 4 * LP)
    w_out[0] = w4.transpose(2, 1, 0, 3).reshape(NH, QB, 4 * LP)


_prep_call = pl.pallas_call(
    _prep_body,
    grid=(BS, NBLK),
    in_specs=[
        pl.BlockSpec((1, QB, DM), lambda b, i: (b, i, 0)),
        pl.BlockSpec((1, QB, DM), lambda b, i: (b, i, 0)),
        pl.BlockSpec((1, QB, NL), lambda b, i: (b, i, 0)),
        pl.BlockSpec((1, QB, NL), lambda b, i: (b, i, 0)),
        pl.BlockSpec((DM, DM), lambda b, i: (0, 0)),
        pl.BlockSpec((1, DM), lambda b, i: (0, 0)),
        pl.BlockSpec((NH * LP, DM), lambda b, i: (0, 0)),
        pl.BlockSpec((1, NH * LP), lambda b, i: (0, 0)),
        pl.BlockSpec((NH * LP, DM), lambda b, i: (0, 0)),
        pl.BlockSpec((1, NH * LP), lambda b, i: (0, 0)),
        pl.BlockSpec((NH * LP, DM), lambda b, i: (0, 0)),
        pl.BlockSpec((1, NH * LP), lambda b, i: (0, 0)),
        pl.BlockSpec((8, NH * LP), lambda b, i: (0, 0)),
        pl.BlockSpec((8, NH * LP), lambda b, i: (0, 0)),
    ],
    out_specs=[
        pl.BlockSpec((1, NH, QB, HD), lambda b, i: (b, 0, i, 0)),
        pl.BlockSpec((1, NH, QB, 4 * LP), lambda b, i: (b, 0, i, 0)),
        pl.BlockSpec((1, NH, QB, 4 * LP), lambda b, i: (b, 0, i, 0)),
    ],
    out_shape=[
        jax.ShapeDtypeStruct((BS, NH, LEN_P, HD), jnp.float32),
        jax.ShapeDtypeStruct((BS, NH, LEN_P, 4 * LP), jnp.int32),
        jax.ShapeDtypeStruct((BS, NH, LEN_P, 4 * LP), jnp.float32),
    ],
)


def _sc_body(value_hbm, idx_hbm, w_hbm, out_hbm,
             plane_v, idx_v, w_v, out_v, sem_pl, sem_in, sem_out):
    cid = lax.axis_index("c")
    sid = lax.axis_index("s")
    wid = sid * 2 + cid
    b = wid // NH
    h = wid % NH
    plane_words = LEN_P * HD
    pltpu.async_copy(
        value_hbm.at[pl.ds(wid * plane_words, plane_words)], plane_v, sem_pl
    ).wait()
    iota = lax.iota(jnp.int32, 16)
    iota_hi = iota + 16

    def chunk_body(ci, carry):
        qoff = ci * QC
        c1 = pltpu.async_copy(idx_hbm.at[b, h, pl.ds(qoff, QC)], idx_v, sem_in)
        c2 = pltpu.async_copy(w_hbm.at[b, h, pl.ds(qoff, QC)], w_v, sem_in)
        c1.wait()
        c2.wait()

        lane_consts = [jnp.full((16,), j, jnp.int32) for j in range(LP)]

        @plsc.parallel_loop(0, QC, unroll=2)
        def q_body(qi):
            accs = [jnp.zeros((16,), jnp.float32) for _ in range(8)]
            for k in range(4):
                ivec = idx_v[qi, pl.ds(k * LP, LP)]
                wvec = w_v[qi, pl.ds(k * LP, LP)]
                for j in range(LP):
                    rv = jnp.take_along_axis(
                        ivec, lane_consts[j], axis=0, mode="promise_in_bounds")
                    wv = jnp.take_along_axis(
                        wvec, lane_consts[j], axis=0, mode="promise_in_bounds")
                    g0 = plsc.load_gather(plane_v, [rv + iota])
                    g1 = plsc.load_gather(plane_v, [rv + iota_hi])
                    accs[2 * k] = accs[2 * k] + wv * g0
                    accs[2 * k + 1] = accs[2 * k + 1] + wv * g1
            out_v[qi, pl.ds(0, 16)] = (accs[0] + accs[2]) + (accs[4] + accs[6])
            out_v[qi, pl.ds(16, 16)] = (accs[1] + accs[3]) + (accs[5] + accs[7])
        pltpu.async_copy(out_v, out_hbm.at[b, h, pl.ds(qoff, QC)], sem_out).wait()
        return carry

    lax.fori_loop(0, NCH, chunk_body, 0)


@functools.cache
def _get_sc_call():
    return pl.kernel(
        _sc_body,
        out_type=jax.ShapeDtypeStruct((BS, NH, LEN_P, HD), jnp.float32),
        mesh=plsc.VectorSubcoreMesh(core_axis_name="c", subcore_axis_name="s"),
        compiler_params=pltpu.CompilerParams(needs_layout_passes=False),
        scratch_types=[
            pltpu.VMEM((LEN_P * HD,), jnp.float32),
            pltpu.VMEM((QC, 4 * LP), jnp.int32),
            pltpu.VMEM((QC, 4 * LP), jnp.float32),
            pltpu.VMEM((QC, HD), jnp.float32),
            pltpu.SemaphoreType.DMA,
            pltpu.SemaphoreType.DMA,
            pltpu.SemaphoreType.DMA,
        ],
    )


def _out_body(a_ref, w_ref, b_ref, o_ref):
    cd = (((1,), (1,)), ((), ()))
    a = a_ref[0]
    acc = b_ref[...]
    w = w_ref[...]
    outs = []
    for h in range(NH):
        wh = w[:, h * HD:(h + 1) * HD]
        outs.append(lax.dot_general(a[h], wh, cd,
                                    preferred_element_type=jnp.float32, precision=lax.Precision.HIGHEST))
    o_ref[0] = ((outs[0] + outs[1]) + (outs[2] + outs[3])
                + (outs[4] + outs[5]) + (outs[6] + outs[7]) + acc)


_out_call = pl.pallas_call(
    _out_body,
    grid=(BS, NBLK),
    in_specs=[
        pl.BlockSpec((1, NH, QB, HD), lambda b, i: (b, 0, i, 0)),
        pl.BlockSpec((DM, DM), lambda b, i: (0, 0)),
        pl.BlockSpec((1, DM), lambda b, i: (0, 0)),
    ],
    out_specs=pl.BlockSpec((1, QB, DM), lambda b, i: (b, i, 0)),
    out_shape=jax.ShapeDtypeStruct((BS, LEN_P, DM), jnp.float32),
)


def kernel(query, reference_points, input_flatten, input_spatial_shapes,
           input_level_start_index, W_off, b_off, W_attn, b_attn,
           W_val, b_val, W_out, b_out):
    del input_spatial_shapes, input_level_start_index
    Wx = W_off[0::2]
    Wy = W_off[1::2]
    bx = b_off[0::2].reshape(1, NH * LP)
    by = b_off[1::2].reshape(1, NH * LP)
    pad = ((0, 0), (0, LEN_P - LEN), (0, 0))
    query_p = jnp.pad(query, pad)
    xf_p = jnp.pad(input_flatten, pad)
    rpx = jnp.pad(reference_points[..., 0], pad)
    rpy = jnp.pad(reference_points[..., 1], pad)
    value, idxs, ws = _prep_call(
        query_p, xf_p, rpx, rpy, W_val, b_val.reshape(1, DM),
        Wx, bx, Wy, by, W_attn, b_attn.reshape(1, NH * LP),
        jnp.asarray(CONST_F), jnp.asarray(CONST_I))
    attn = _get_sc_call()(value.reshape(-1), idxs, ws)
    return _out_call(attn, W_out, b_out.reshape(1, DM))[:, :LEN]


# speedup vs baseline: 3020.4581x; 1.1124x over previous
import functools

import numpy as np
import jax
import jax.numpy as jnp
from jax import lax
from jax.experimental import pallas as pl
from jax.experimental.pallas import tpu as pltpu
from jax.experimental.pallas import tpu_sc as plsc

NH = 8
NL = 4
NP = 4
DM = 256
HD = 32
LP = NL * NP
LVL_SHAPES = [(48, 48), (24, 24), (12, 12), (6, 6)]
LEN = sum(h * w for h, w in LVL_SHAPES)
LEN_P = 3072
BS = 4
QB = 768
NBLK = LEN_P // QB
QC = 48
NCH = LEN_P // QC

_l_of_col = (np.arange(NH * LP) // NP) % NL
_W_f = np.array([w for h, w in LVL_SHAPES], np.float32)
_H_f = np.array([h for h, w in LVL_SHAPES], np.float32)
_hw = np.array([h * w for h, w in LVL_SHAPES], np.int64)
_start = np.concatenate([[0], np.cumsum(_hw)[:-1]]).astype(np.int32)
_cf = np.zeros((8, NH * LP), np.float32)
_cf[_l_of_col, np.arange(NH * LP)] = 1.0
_cf[4] = _W_f[_l_of_col]
_cf[5] = _H_f[_l_of_col]
CONST_F = _cf
_ci = np.zeros((8, NH * LP), np.int32)
_ci[0] = _W_f[_l_of_col].astype(np.int32)
_ci[1] = _H_f[_l_of_col].astype(np.int32)
_ci[2] = _start[_l_of_col]
CONST_I = _ci


def _prep_body(q_ref, x_ref, rpx_ref, rpy_ref, wval_ref, bval_ref,
               wx_ref, bx_ref, wy_ref, by_ref, wa_ref, ba_ref,
               cf_ref, ci_ref, val_out, idx_out, w_out):
    f32 = jnp.float32
    cd = (((1,), (1,)), ((), ()))
    q = q_ref[0]
    x_in = x_ref[0]
    val_out[0] = (lax.dot_general(x_in, wval_ref[...], cd,
                                  preferred_element_type=f32,
                                  precision=lax.Precision.HIGHEST)
                  + bval_ref[...])
    offx = lax.dot_general(q, wx_ref[...], cd, preferred_element_type=f32, precision=lax.Precision.HIGHEST) + bx_ref[...]
    offy = lax.dot_general(q, wy_ref[...], cd, preferred_element_type=f32, precision=lax.Precision.HIGHEST) + by_ref[...]
    logits = lax.dot_general(q, wa_ref[...], cd, preferred_element_type=f32, precision=lax.Precision.HIGHEST) + ba_ref[...]
    l3 = logits.reshape(QB, NH, LP)
    l3 = l3 - jnp.max(l3, axis=2, keepdims=True)
    e = jnp.exp(l3)
    aw = (e / jnp.sum(e, axis=2, keepdims=True)).reshape(QB, NH * LP)

    sel = cf_ref[0:NL]
    col_w = cf_ref[4:5]
    col_h = cf_ref[5:6]
    col_wi = ci_ref[0:1]
    col_hi = ci_ref[1:2]
    col_start = ci_ref[2:3]
    cd0 = (((1,), (0,)), ((), ()))
    rx = lax.dot_general(rpx_ref[0], sel, cd0, preferred_element_type=f32, precision=lax.Precision.HIGHEST)
    ry = lax.dot_general(rpy_ref[0], sel, cd0, preferred_element_type=f32, precision=lax.Precision.HIGHEST)
    x = rx * col_w + offx - 0.5
    y = ry * col_h + offy - 0.5
    x0f = jnp.floor(x)
    y0f = jnp.floor(y)
    fx = x - x0f
    fy = y - y0f
    ix0 = x0f.astype(jnp.int32)
    iy0 = y0f.astype(jnp.int32)
    one = jnp.float32(1.0)
    vx0 = ((x0f >= 0) & (x0f <= col_w - 1)).astype(f32)
    vx1 = ((x0f + 1 >= 0) & (x0f + 1 <= col_w - 1)).astype(f32)
    vy0 = ((y0f >= 0) & (y0f <= col_h - 1)).astype(f32)
    vy1 = ((y0f + 1 >= 0) & (y0f + 1 <= col_h - 1)).astype(f32)
    cx0 = jnp.clip(ix0, 0, col_wi - 1)
    cx1 = jnp.clip(ix0 + 1, 0, col_wi - 1)
    cy0 = jnp.clip(iy0, 0, col_hi - 1)
    cy1 = jnp.clip(iy0 + 1, 0, col_hi - 1)
    r00 = (col_start + cy0 * col_wi + cx0) * HD
    r01 = (col_start + cy0 * col_wi + cx1) * HD
    r10 = (col_start + cy1 * col_wi + cx0) * HD
    r11 = (col_start + cy1 * col_wi + cx1) * HD
    w00 = (one - fx) * (one - fy) * vx0 * vy0 * aw
    w01 = fx * (one - fy) * vx1 * vy0 * aw
    w10 = (one - fx) * fy * vx0 * vy1 * aw
    w11 = fx * fy * vx1 * vy1 * aw
    idx_out[0, :, 0, :] = r00
    idx_out[0, :, 1, :] = r01
    idx_out[0, :, 2, :] = r10
    idx_out[0, :, 3, :] = r11
    w_out[0, :, 0, :] = w00
    w_out[0, :, 1, :] = w01
    w_out[0, :, 2, :] = w10
    w_out[0, :, 3, :] = w11


_prep_call = pl.pallas_call(
    _prep_body,
    grid=(BS, NBLK),
    in_specs=[
        pl.BlockSpec((1, QB, DM), lambda b, i: (b, i, 0)),
        pl.BlockSpec((1, QB, DM), lambda b, i: (b, i, 0)),
        pl.BlockSpec((1, QB, NL), lambda b, i: (b, i, 0)),
        pl.BlockSpec((1, QB, NL), lambda b, i: (b, i, 0)),
        pl.BlockSpec((DM, DM), lambda b, i: (0, 0)),
        pl.BlockSpec((1, DM), lambda b, i: (0, 0)),
        pl.BlockSpec((NH * LP, DM), lambda b, i: (0, 0)),
        pl.BlockSpec((1, NH * LP), lambda b, i: (0, 0)),
        pl.BlockSpec((NH * LP, DM), lambda b, i: (0, 0)),
        pl.BlockSpec((1, NH * LP), lambda b, i: (0, 0)),
        pl.BlockSpec((NH * LP, DM), lambda b, i: (0, 0)),
        pl.BlockSpec((1, NH * LP), lambda b, i: (0, 0)),
        pl.BlockSpec((8, NH * LP), lambda b, i: (0, 0)),
        pl.BlockSpec((8, NH * LP), lambda b, i: (0, 0)),
    ],
    out_specs=[
        pl.BlockSpec((1, QB, DM), lambda b, i: (b, i, 0)),
        pl.BlockSpec((1, QB, 4, NH * LP), lambda b, i: (b, i, 0, 0)),
        pl.BlockSpec((1, QB, 4, NH * LP), lambda b, i: (b, i, 0, 0)),
    ],
    out_shape=[
        jax.ShapeDtypeStruct((BS, LEN_P, DM), jnp.float32),
        jax.ShapeDtypeStruct((BS, LEN_P, 4, NH * LP), jnp.int32),
        jax.ShapeDtypeStruct((BS, LEN_P, 4, NH * LP), jnp.float32),
    ],
)


def _sc_body(value_hbm, idx_hbm, w_hbm, out_hbm,
             plane_v, idx_v, w_v, out_v, sem_pl, sem_in, sem_out):
    cid = lax.axis_index("c")
    sid = lax.axis_index("s")
    wid = sid * 2 + cid
    b = wid // NH
    h = wid % NH
    plane_words = LEN_P * HD
    pltpu.async_copy(
        value_hbm.at[pl.ds(wid * plane_words, plane_words)], plane_v, sem_pl
    ).wait()
    iota = lax.iota(jnp.int32, 16)
    iota_hi = iota + 16

    def chunk_body(ci, carry):
        qoff = ci * QC
        c1 = pltpu.async_copy(idx_hbm.at[b, h, pl.ds(qoff, QC)], idx_v, sem_in)
        c2 = pltpu.async_copy(w_hbm.at[b, h, pl.ds(qoff, QC)], w_v, sem_in)
        c1.wait()
        c2.wait()

        lane_consts = [jnp.full((16,), j, jnp.int32) for j in range(LP)]

        @plsc.parallel_loop(0, QC, unroll=2)
        def q_body(qi):
            accs = [jnp.zeros((16,), jnp.float32) for _ in range(8)]
            for k in range(4):
                ivec = idx_v[qi, pl.ds(k * LP, LP)]
                wvec = w_v[qi, pl.ds(k * LP, LP)]
                for j in range(LP):
                    rv = jnp.take_along_axis(
                        ivec, lane_consts[j], axis=0, mode="promise_in_bounds")
                    wv = jnp.take_along_axis(
                        wvec, lane_consts[j], axis=0, mode="promise_in_bounds")
                    g0 = plsc.load_gather(plane_v, [rv + iota])
                    g1 = plsc.load_gather(plane_v, [rv + iota_hi])
                    accs[2 * k] = accs[2 * k] + wv * g0
                    accs[2 * k + 1] = accs[2 * k + 1] + wv * g1
            out_v[qi, pl.ds(0, 16)] = (accs[0] + accs[2]) + (accs[4] + accs[6])
            out_v[qi, pl.ds(16, 16)] = (accs[1] + accs[3]) + (accs[5] + accs[7])
        pltpu.async_copy(out_v, out_hbm.at[b, h, pl.ds(qoff, QC)], sem_out).wait()
        return carry

    lax.fori_loop(0, NCH, chunk_body, 0)


@functools.cache
def _get_sc_call():
    return pl.kernel(
        _sc_body,
        out_type=jax.ShapeDtypeStruct((BS, NH, LEN_P, HD), jnp.float32),
        mesh=plsc.VectorSubcoreMesh(core_axis_name="c", subcore_axis_name="s"),
        compiler_params=pltpu.CompilerParams(needs_layout_passes=False),
        scratch_types=[
            pltpu.VMEM((LEN_P * HD,), jnp.float32),
            pltpu.VMEM((QC, 4 * LP), jnp.int32),
            pltpu.VMEM((QC, 4 * LP), jnp.float32),
            pltpu.VMEM((QC, HD), jnp.float32),
            pltpu.SemaphoreType.DMA,
            pltpu.SemaphoreType.DMA,
            pltpu.SemaphoreType.DMA,
        ],
    )


def _out_body(a_ref, w_ref, b_ref, o_ref):
    cd = (((1,), (1,)), ((), ()))
    a = a_ref[0]
    acc = b_ref[...]
    w = w_ref[...]
    outs = []
    for h in range(NH):
        wh = w[:, h * HD:(h + 1) * HD]
        outs.append(lax.dot_general(a[h], wh, cd,
                                    preferred_element_type=jnp.float32, precision=lax.Precision.HIGHEST))
    o_ref[0] = ((outs[0] + outs[1]) + (outs[2] + outs[3])
                + (outs[4] + outs[5]) + (outs[6] + outs[7]) + acc)


_out_call = pl.pallas_call(
    _out_body,
    grid=(BS, NBLK),
    in_specs=[
        pl.BlockSpec((1, NH, QB, HD), lambda b, i: (b, 0, i, 0)),
        pl.BlockSpec((DM, DM), lambda b, i: (0, 0)),
        pl.BlockSpec((1, DM), lambda b, i: (0, 0)),
    ],
    out_specs=pl.BlockSpec((1, QB, DM), lambda b, i: (b, i, 0)),
    out_shape=jax.ShapeDtypeStruct((BS, LEN_P, DM), jnp.float32),
)


def kernel(query, reference_points, input_flatten, input_spatial_shapes,
           input_level_start_index, W_off, b_off, W_attn, b_attn,
           W_val, b_val, W_out, b_out):
    del input_spatial_shapes, input_level_start_index
    Wx = W_off[0::2]
    Wy = W_off[1::2]
    bx = b_off[0::2].reshape(1, NH * LP)
    by = b_off[1::2].reshape(1, NH * LP)
    pad = ((0, 0), (0, LEN_P - LEN), (0, 0))
    query_p = jnp.pad(query, pad)
    xf_p = jnp.pad(input_flatten, pad)
    rpx = jnp.pad(reference_points[..., 0], pad)
    rpy = jnp.pad(reference_points[..., 1], pad)
    value, idxs, ws = _prep_call(
        query_p, xf_p, rpx, rpy, W_val, b_val.reshape(1, DM),
        Wx, bx, Wy, by, W_attn, b_attn.reshape(1, NH * LP),
        jnp.asarray(CONST_F), jnp.asarray(CONST_I))
    value_sc = value.reshape(BS, LEN_P, NH, HD).transpose(0, 2, 1, 3).reshape(-1)
    idx_sc = (idxs.reshape(BS, LEN_P, 4, NH, LP)
              .transpose(0, 3, 1, 2, 4).reshape(BS, NH, LEN_P, 4 * LP))
    w_sc = (ws.reshape(BS, LEN_P, 4, NH, LP)
            .transpose(0, 3, 1, 2, 4).reshape(BS, NH, LEN_P, 4 * LP))
    attn = _get_sc_call()(value_sc, idx_sc, w_sc)
    return _out_call(attn, W_out, b_out.reshape(1, DM))[:, :LEN]


# speedup vs baseline: 3324.5709x; 1.1007x over previous
import functools

import numpy as np
import jax
import jax.numpy as jnp
from jax import lax
from jax.experimental import pallas as pl
from jax.experimental.pallas import tpu as pltpu
from jax.experimental.pallas import tpu_sc as plsc

NH = 8
NL = 4
NP = 4
DM = 256
HD = 32
LP = NL * NP
LVL_SHAPES = [(48, 48), (24, 24), (12, 12), (6, 6)]
LEN = sum(h * w for h, w in LVL_SHAPES)
LEN_P = 3072
BS = 4
QB = 768
NBLK = LEN_P // QB
QC = 32
NCH = LEN_P // QC

_l_of_col = (np.arange(NH * LP) // NP) % NL
_W_f = np.array([w for h, w in LVL_SHAPES], np.float32)
_H_f = np.array([h for h, w in LVL_SHAPES], np.float32)
_hw = np.array([h * w for h, w in LVL_SHAPES], np.int64)
_start = np.concatenate([[0], np.cumsum(_hw)[:-1]]).astype(np.int32)
_cf = np.zeros((8, NH * LP), np.float32)
_cf[_l_of_col, np.arange(NH * LP)] = 1.0
_cf[4] = _W_f[_l_of_col]
_cf[5] = _H_f[_l_of_col]
CONST_F = _cf
_ci = np.zeros((8, NH * LP), np.int32)
_ci[0] = _W_f[_l_of_col].astype(np.int32)
_ci[1] = _H_f[_l_of_col].astype(np.int32)
_ci[2] = _start[_l_of_col]
CONST_I = _ci


def _prep_body(q_ref, x_ref, rpx_ref, rpy_ref, wval_ref, bval_ref,
               wx_ref, bx_ref, wy_ref, by_ref, wa_ref, ba_ref,
               cf_ref, ci_ref, val_out, idx_out, w_out):
    f32 = jnp.float32
    cd = (((1,), (1,)), ((), ()))
    q = q_ref[0]
    x_in = x_ref[0]
    val_out[0] = (lax.dot_general(x_in, wval_ref[...], cd,
                                  preferred_element_type=f32,
                                  precision=lax.Precision.HIGHEST)
                  + bval_ref[...])
    offx = lax.dot_general(q, wx_ref[...], cd, preferred_element_type=f32, precision=lax.Precision.HIGHEST) + bx_ref[...]
    offy = lax.dot_general(q, wy_ref[...], cd, preferred_element_type=f32, precision=lax.Precision.HIGHEST) + by_ref[...]
    logits = lax.dot_general(q, wa_ref[...], cd, preferred_element_type=f32, precision=lax.Precision.HIGHEST) + ba_ref[...]
    l3 = logits.reshape(QB, NH, LP)
    l3 = l3 - jnp.max(l3, axis=2, keepdims=True)
    e = jnp.exp(l3)
    aw = (e / jnp.sum(e, axis=2, keepdims=True)).reshape(QB, NH * LP)

    sel = cf_ref[0:NL]
    col_w = cf_ref[4:5]
    col_h = cf_ref[5:6]
    col_wi = ci_ref[0:1]
    col_hi = ci_ref[1:2]
    col_start = ci_ref[2:3]
    cd0 = (((1,), (0,)), ((), ()))
    rx = lax.dot_general(rpx_ref[0], sel, cd0, preferred_element_type=f32, precision=lax.Precision.HIGHEST)
    ry = lax.dot_general(rpy_ref[0], sel, cd0, preferred_element_type=f32, precision=lax.Precision.HIGHEST)
    x = rx * col_w + offx - 0.5
    y = ry * col_h + offy - 0.5
    x0f = jnp.floor(x)
    y0f = jnp.floor(y)
    fx = x - x0f
    fy = y - y0f
    ix0 = x0f.astype(jnp.int32)
    iy0 = y0f.astype(jnp.int32)
    one = jnp.float32(1.0)
    vx0 = ((x0f >= 0) & (x0f <= col_w - 1)).astype(f32)
    vx1 = ((x0f + 1 >= 0) & (x0f + 1 <= col_w - 1)).astype(f32)
    vy0 = ((y0f >= 0) & (y0f <= col_h - 1)).astype(f32)
    vy1 = ((y0f + 1 >= 0) & (y0f + 1 <= col_h - 1)).astype(f32)
    cx0 = jnp.clip(ix0, 0, col_wi - 1)
    cx1 = jnp.clip(ix0 + 1, 0, col_wi - 1)
    cy0 = jnp.clip(iy0, 0, col_hi - 1)
    cy1 = jnp.clip(iy0 + 1, 0, col_hi - 1)
    r00 = (col_start + cy0 * col_wi + cx0) * HD
    r01 = (col_start + cy0 * col_wi + cx1) * HD
    r10 = (col_start + cy1 * col_wi + cx0) * HD
    r11 = (col_start + cy1 * col_wi + cx1) * HD
    w00 = (one - fx) * (one - fy) * vx0 * vy0 * aw
    w01 = fx * (one - fy) * vx1 * vy0 * aw
    w10 = (one - fx) * fy * vx0 * vy1 * aw
    w11 = fx * fy * vx1 * vy1 * aw
    idx_out[0, :, 0, :] = r00
    idx_out[0, :, 1, :] = r01
    idx_out[0, :, 2, :] = r10
    idx_out[0, :, 3, :] = r11
    w_out[0, :, 0, :] = w00
    w_out[0, :, 1, :] = w01
    w_out[0, :, 2, :] = w10
    w_out[0, :, 3, :] = w11


_prep_call = pl.pallas_call(
    _prep_body,
    grid=(BS, NBLK),
    in_specs=[
        pl.BlockSpec((1, QB, DM), lambda b, i: (b, i, 0)),
        pl.BlockSpec((1, QB, DM), lambda b, i: (b, i, 0)),
        pl.BlockSpec((1, QB, NL), lambda b, i: (b, i, 0)),
        pl.BlockSpec((1, QB, NL), lambda b, i: (b, i, 0)),
        pl.BlockSpec((DM, DM), lambda b, i: (0, 0)),
        pl.BlockSpec((1, DM), lambda b, i: (0, 0)),
        pl.BlockSpec((NH * LP, DM), lambda b, i: (0, 0)),
        pl.BlockSpec((1, NH * LP), lambda b, i: (0, 0)),
        pl.BlockSpec((NH * LP, DM), lambda b, i: (0, 0)),
        pl.BlockSpec((1, NH * LP), lambda b, i: (0, 0)),
        pl.BlockSpec((NH * LP, DM), lambda b, i: (0, 0)),
        pl.BlockSpec((1, NH * LP), lambda b, i: (0, 0)),
        pl.BlockSpec((8, NH * LP), lambda b, i: (0, 0)),
        pl.BlockSpec((8, NH * LP), lambda b, i: (0, 0)),
    ],
    out_specs=[
        pl.BlockSpec((1, QB, DM), lambda b, i: (b, i, 0)),
        pl.BlockSpec((1, QB, 4, NH * LP), lambda b, i: (b, i, 0, 0)),
        pl.BlockSpec((1, QB, 4, NH * LP), lambda b, i: (b, i, 0, 0)),
    ],
    out_shape=[
        jax.ShapeDtypeStruct((BS, LEN_P, DM), jnp.float32),
        jax.ShapeDtypeStruct((BS, LEN_P, 4, NH * LP), jnp.int32),
        jax.ShapeDtypeStruct((BS, LEN_P, 4, NH * LP), jnp.float32),
    ],
)


def _sc_body(value_hbm, idx_hbm, w_hbm, out_hbm,
             plane_v, idx_v, w_v, out_v, sem_pl, sem_in, sem_out):
    cid = lax.axis_index("c")
    sid = lax.axis_index("s")
    wid = sid * 2 + cid
    b = wid // NH
    h = wid % NH
    plane_words = LEN_P * HD
    pltpu.async_copy(
        value_hbm.at[pl.ds(wid * plane_words, plane_words)], plane_v, sem_pl
    ).wait()
    iota = lax.iota(jnp.int32, 16)
    iota_hi = iota + 16

    lane_consts = [jnp.full((16,), j, jnp.int32) for j in range(LP)]

    def fire_in(ci, sl):
        pltpu.async_copy(
            idx_hbm.at[b, h, pl.ds(ci * QC, QC)], idx_v.at[sl], sem_in)
        pltpu.async_copy(
            w_hbm.at[b, h, pl.ds(ci * QC, QC)], w_v.at[sl], sem_in)

    def drain_in(sl):
        pltpu.make_async_copy(
            idx_hbm.at[b, h, pl.ds(0, QC)], idx_v.at[sl], sem_in).wait()
        pltpu.make_async_copy(
            w_hbm.at[b, h, pl.ds(0, QC)], w_v.at[sl], sem_in).wait()

    def drain_out(sl):
        pltpu.make_async_copy(
            out_v.at[sl], out_hbm.at[b, h, pl.ds(0, QC)], sem_out).wait()

    fire_in(0, 0)
    fire_in(1, 1)

    def pair_body(pi, carry):
        for sl in range(2):
            ci = 2 * pi + sl
            drain_in(sl)

            @pl.when(pi >= 1)
            def _():
                drain_out(sl)

            @plsc.parallel_loop(0, QC, unroll=2)
            def q_body(qi):
                accs = [jnp.zeros((16,), jnp.float32) for _ in range(8)]
                for k in range(4):
                    ivec = idx_v[sl, qi, pl.ds(k * LP, LP)]
                    wvec = w_v[sl, qi, pl.ds(k * LP, LP)]
                    for j in range(LP):
                        rv = jnp.take_along_axis(
                            ivec, lane_consts[j], axis=0,
                            mode="promise_in_bounds")
                        wv = jnp.take_along_axis(
                            wvec, lane_consts[j], axis=0,
                            mode="promise_in_bounds")
                        g0 = plsc.load_gather(plane_v, [rv + iota])
                        g1 = plsc.load_gather(plane_v, [rv + iota_hi])
                        accs[2 * k] = accs[2 * k] + wv * g0
                        accs[2 * k + 1] = accs[2 * k + 1] + wv * g1
                out_v[sl, qi, pl.ds(0, 16)] = (
                    (accs[0] + accs[2]) + (accs[4] + accs[6]))
                out_v[sl, qi, pl.ds(16, 16)] = (
                    (accs[1] + accs[3]) + (accs[5] + accs[7]))

            pltpu.async_copy(
                out_v.at[sl], out_hbm.at[b, h, pl.ds(ci * QC, QC)], sem_out)

            @pl.when(pi < NCH // 2 - 1)
            def _():
                fire_in(ci + 2, sl)
        return carry

    lax.fori_loop(0, NCH // 2, pair_body, 0)
    drain_out(0)
    drain_out(1)


@functools.cache
def _get_sc_call():
    return pl.kernel(
        _sc_body,
        out_type=jax.ShapeDtypeStruct((BS, NH, LEN_P, HD), jnp.float32),
        mesh=plsc.VectorSubcoreMesh(core_axis_name="c", subcore_axis_name="s"),
        compiler_params=pltpu.CompilerParams(needs_layout_passes=False),
        scratch_types=[
            pltpu.VMEM((LEN_P * HD,), jnp.float32),
            pltpu.VMEM((2, QC, 4 * LP), jnp.int32),
            pltpu.VMEM((2, QC, 4 * LP), jnp.float32),
            pltpu.VMEM((2, QC, HD), jnp.float32),
            pltpu.SemaphoreType.DMA,
            pltpu.SemaphoreType.DMA,
            pltpu.SemaphoreType.DMA,
        ],
    )


def _out_body(a_ref, w_ref, b_ref, o_ref):
    cd = (((1,), (1,)), ((), ()))
    a = a_ref[0]
    acc = b_ref[...]
    w = w_ref[...]
    outs = []
    for h in range(NH):
        wh = w[:, h * HD:(h + 1) * HD]
        outs.append(lax.dot_general(a[h], wh, cd,
                                    preferred_element_type=jnp.float32, precision=lax.Precision.HIGHEST))
    o_ref[0] = ((outs[0] + outs[1]) + (outs[2] + outs[3])
                + (outs[4] + outs[5]) + (outs[6] + outs[7]) + acc)


_out_call = pl.pallas_call(
    _out_body,
    grid=(BS, NBLK),
    in_specs=[
        pl.BlockSpec((1, NH, QB, HD), lambda b, i: (b, 0, i, 0)),
        pl.BlockSpec((DM, DM), lambda b, i: (0, 0)),
        pl.BlockSpec((1, DM), lambda b, i: (0, 0)),
    ],
    out_specs=pl.BlockSpec((1, QB, DM), lambda b, i: (b, i, 0)),
    out_shape=jax.ShapeDtypeStruct((BS, LEN_P, DM), jnp.float32),
)


def kernel(query, reference_points, input_flatten, input_spatial_shapes,
           input_level_start_index, W_off, b_off, W_attn, b_attn,
           W_val, b_val, W_out, b_out):
    del input_spatial_shapes, input_level_start_index
    Wx = W_off[0::2]
    Wy = W_off[1::2]
    bx = b_off[0::2].reshape(1, NH * LP)
    by = b_off[1::2].reshape(1, NH * LP)
    pad = ((0, 0), (0, LEN_P - LEN), (0, 0))
    query_p = jnp.pad(query, pad)
    xf_p = jnp.pad(input_flatten, pad)
    rpx = jnp.pad(reference_points[..., 0], pad)
    rpy = jnp.pad(reference_points[..., 1], pad)
    value, idxs, ws = _prep_call(
        query_p, xf_p, rpx, rpy, W_val, b_val.reshape(1, DM),
        Wx, bx, Wy, by, W_attn, b_attn.reshape(1, NH * LP),
        jnp.asarray(CONST_F), jnp.asarray(CONST_I))
    value_sc = value.reshape(BS, LEN_P, NH, HD).transpose(0, 2, 1, 3).reshape(-1)
    idx_sc = (idxs.reshape(BS, LEN_P, 4, NH, LP)
              .transpose(0, 3, 1, 2, 4).reshape(BS, NH, LEN_P, 4 * LP))
    w_sc = (ws.reshape(BS, LEN_P, 4, NH, LP)
            .transpose(0, 3, 1, 2, 4).reshape(BS, NH, LEN_P, 4 * LP))
    attn = _get_sc_call()(value_sc, idx_sc, w_sc)
    return _out_call(attn, W_out, b_out.reshape(1, DM))[:, :LEN]
